# Initial kernel scaffold; baseline (speedup 1.0000x reference)
import jax
import jax.numpy as jnp
from jax.experimental import pallas as pl


def kernel(ht_idx, qual_idx, ent_embedding, rel_embedding):
    raise NotImplementedError("write your pallas kernel here")


# speedup vs baseline: 2.3173x; 2.3173x over previous
import functools

import jax
import jax.numpy as jnp
from jax import lax
from jax.experimental import pallas as pl
from jax.experimental.pallas import tpu as pltpu
from jax.experimental.pallas import tpu_sc as plsc

NUM_ENT = 100000
NUM_REL = 500
D = 128
B = 4096
T = 5
QUAL = 8

NC = 2
NS = 16
NW = NC * NS

CHUNK = 128

N_HT = B * T * 2
N_Q = B * T * (QUAL // 2)

HT_CHUNKS = N_HT // (NW * CHUNK)
Q_CHUNKS = N_Q // (NW * CHUNK)

GROUPS = CHUNK // 16

_LANE = None


def _fix_padding(idx_ref, j, rows_ref):
    zeros16 = jnp.zeros((16,), jnp.float32)

    def group_body(g, carry):
        idx16 = idx_ref[pl.ds(j * CHUNK + g * 16, 16)]
        is_zero = idx16 == 0
        nzero = plsc.all_reduce_population_count(is_zero)

        @pl.when(nzero[0] > 0)
        def _():
            lane = lax.iota(jnp.int32, 16)
            for jj in range(16):
                nj = plsc.all_reduce_population_count(is_zero & (lane == jj))

                @pl.when(nj[0] > 0)
                def _():
                    row = g * 16 + jj
                    for c in range(D // 16):
                        rows_ref[row, pl.ds(c * 16, 16)] = zeros16

        return carry

    lax.fori_loop(0, GROUPS, group_body, 0)


def _body(ent_hbm, rel_hbm, hti_hbm, qri_hbm, qei_hbm,
          out_ht, out_qr, out_qe, idx_v, rows_v, sem):
    wid = lax.axis_index("s") * NC + lax.axis_index("c")

    for (tab, idx_hbm, out_hbm, nch) in (
        (ent_hbm, hti_hbm, out_ht, HT_CHUNKS),
        (rel_hbm, qri_hbm, out_qr, Q_CHUNKS),
        (ent_hbm, qei_hbm, out_qe, Q_CHUNKS),
    ):
        pw = nch * CHUNK
        pltpu.sync_copy(idx_hbm.at[pl.ds(wid * pw, pw)],
                        idx_v.at[pl.ds(0, pw)])

        def chunk_body(j, carry, tab=tab, out_hbm=out_hbm, nch=nch):
            pltpu.async_copy(tab.at[idx_v.at[pl.ds(j * CHUNK, CHUNK)]],
                             rows_v, sem).wait()
            _fix_padding(idx_v, j, rows_v)
            base = (wid * nch + j) * CHUNK
            pltpu.sync_copy(rows_v, out_hbm.at[pl.ds(base, CHUNK)])
            return carry

        lax.fori_loop(0, nch, chunk_body, 0)


@jax.jit
def _run(ht_flat, qrel_flat, qent_flat, ent_embedding, rel_embedding):
    mesh = plsc.VectorSubcoreMesh(core_axis_name="c", subcore_axis_name="s",
                                  num_cores=NC, num_subcores=NS)
    out_type = (
        jax.ShapeDtypeStruct((N_HT, D), jnp.float32),
        jax.ShapeDtypeStruct((N_Q, D), jnp.float32),
        jax.ShapeDtypeStruct((N_Q, D), jnp.float32),
    )
    scratch = [
        pltpu.VMEM((Q_CHUNKS * CHUNK,), jnp.int32),
        pltpu.VMEM((CHUNK, D), jnp.float32),
        pltpu.SemaphoreType.DMA,
    ]
    f = pl.kernel(_body, out_type=out_type, mesh=mesh, scratch_types=scratch,
                  compiler_params=pltpu.CompilerParams(
                      needs_layout_passes=False))
    return f(ent_embedding, rel_embedding, ht_flat, qrel_flat, qent_flat)


def kernel(ht_idx, qual_idx, ent_embedding, rel_embedding):
    ht_flat = ht_idx.astype(jnp.int32).reshape(N_HT)
    qual = qual_idx.astype(jnp.int32)
    qrel_flat = qual[:, :, ::2].reshape(N_Q)
    qent_flat = qual[:, :, 1::2].reshape(N_Q)
    out_ht, out_qr, out_qe = _run(ht_flat, qrel_flat, qent_flat,
                                  ent_embedding, rel_embedding)
    return (out_ht.reshape(B, T, 2, D),
            out_qr.reshape(B, T, QUAL // 2, D),
            out_qe.reshape(B, T, QUAL // 2, D))


# speedup vs baseline: 2.4302x; 1.0487x over previous
import functools

import jax
import jax.numpy as jnp
from jax import lax
from jax.experimental import pallas as pl
from jax.experimental.pallas import tpu as pltpu
from jax.experimental.pallas import tpu_sc as plsc

NUM_ENT = 100000
NUM_REL = 500
D = 128
B = 4096
T = 5
QUAL = 8

NC = 2
NS = 16
NW = NC * NS

CHUNK = 128

N_HT = B * T * 2
N_Q = B * T * (QUAL // 2)

HT_CHUNKS = N_HT // (NW * CHUNK)
Q_CHUNKS = N_Q // (NW * CHUNK)

GROUPS = CHUNK // 16

_LANE = None


def _fix_padding(idx_ref, j, rows_ref):
    zeros16 = jnp.zeros((16,), jnp.float32)

    def group_body(g, carry):
        idx16 = idx_ref[pl.ds(j * CHUNK + g * 16, 16)]
        is_zero = idx16 == 0
        nzero = plsc.all_reduce_population_count(is_zero)

        @pl.when(nzero[0] > 0)
        def _():
            lane = lax.iota(jnp.int32, 16)
            for jj in range(16):
                nj = plsc.all_reduce_population_count(is_zero & (lane == jj))

                @pl.when(nj[0] > 0)
                def _():
                    row = g * 16 + jj
                    for c in range(D // 16):
                        rows_ref[row, pl.ds(c * 16, 16)] = zeros16

        return carry

    lax.fori_loop(0, GROUPS, group_body, 0)


def _body(ent_hbm, rel_hbm, hti_hbm, qri_hbm, qei_hbm,
          out_ht, out_qr, out_qe, idx_v, rows_v, sem_g, sem_w):
    wid = lax.axis_index("s") * NC + lax.axis_index("c")

    for (tab, idx_hbm, out_hbm, nch) in (
        (ent_hbm, hti_hbm, out_ht, HT_CHUNKS),
        (rel_hbm, qri_hbm, out_qr, Q_CHUNKS),
        (ent_hbm, qei_hbm, out_qe, Q_CHUNKS),
    ):
        pw = nch * CHUNK
        pltpu.sync_copy(idx_hbm.at[pl.ds(wid * pw, pw)],
                        idx_v.at[pl.ds(0, pw)])

        def gather(j, b, tab=tab):
            pltpu.async_copy(tab.at[idx_v.at[pl.ds(j * CHUNK, CHUNK)]],
                             rows_v.at[b], sem_g)

        def start_write(j, b, out_hbm=out_hbm, nch=nch):
            base = (wid * nch + j) * CHUNK
            pltpu.async_copy(rows_v.at[b], out_hbm.at[pl.ds(base, CHUNK)],
                             sem_w)

        def wait_write(j, b, out_hbm=out_hbm, nch=nch):
            base = (wid * nch + j) * CHUNK
            pltpu.make_async_copy(rows_v.at[b],
                                  out_hbm.at[pl.ds(base, CHUNK)],
                                  sem_w).wait()

        def wait_gather(j, b, tab=tab):
            pltpu.make_async_copy(tab.at[idx_v.at[pl.ds(j * CHUNK, CHUNK)]],
                                  rows_v.at[b], sem_g).wait()

        gather(0, 0)

        def chunk_body(j, carry):
            b = j % 2
            wait_gather(j, b)

            @pl.when(j >= 1)
            def _():
                wait_write(j - 1, 1 - b)

            @pl.when(j + 1 < nch)
            def _():
                gather(j + 1, 1 - b)

            _fix_padding(idx_v, j, rows_v.at[b])
            start_write(j, b)
            return carry

        lax.fori_loop(0, nch, chunk_body, 0)
        wait_write(nch - 1, (nch - 1) % 2)


@jax.jit
def _run(ht_flat, qrel_flat, qent_flat, ent_embedding, rel_embedding):
    mesh = plsc.VectorSubcoreMesh(core_axis_name="c", subcore_axis_name="s",
                                  num_cores=NC, num_subcores=NS)
    out_type = (
        jax.ShapeDtypeStruct((N_HT, D), jnp.float32),
        jax.ShapeDtypeStruct((N_Q, D), jnp.float32),
        jax.ShapeDtypeStruct((N_Q, D), jnp.float32),
    )
    scratch = [
        pltpu.VMEM((Q_CHUNKS * CHUNK,), jnp.int32),
        pltpu.VMEM((2, CHUNK, D), jnp.float32),
        pltpu.SemaphoreType.DMA,
        pltpu.SemaphoreType.DMA,
    ]
    f = pl.kernel(_body, out_type=out_type, mesh=mesh, scratch_types=scratch,
                  compiler_params=pltpu.CompilerParams(
                      needs_layout_passes=False))
    return f(ent_embedding, rel_embedding, ht_flat, qrel_flat, qent_flat)


def kernel(ht_idx, qual_idx, ent_embedding, rel_embedding):
    ht_flat = ht_idx.astype(jnp.int32).reshape(N_HT)
    qual = qual_idx.astype(jnp.int32)
    qrel_flat = qual[:, :, ::2].reshape(N_Q)
    qent_flat = qual[:, :, 1::2].reshape(N_Q)
    out_ht, out_qr, out_qe = _run(ht_flat, qrel_flat, qent_flat,
                                  ent_embedding, rel_embedding)
    return (out_ht.reshape(B, T, 2, D),
            out_qr.reshape(B, T, QUAL // 2, D),
            out_qe.reshape(B, T, QUAL // 2, D))


# speedup vs baseline: 2.5037x; 1.0303x over previous
import functools

import jax
import jax.numpy as jnp
from jax import lax
from jax.experimental import pallas as pl
from jax.experimental.pallas import tpu as pltpu
from jax.experimental.pallas import tpu_sc as plsc

NUM_ENT = 100000
NUM_REL = 500
D = 128
B = 4096
T = 5
QUAL = 8

NC = 2
NS = 16
NW = NC * NS

CHUNK = 128
NBUF = 4

N_HT = B * T * 2
N_Q = B * T * (QUAL // 2)

HT_CHUNKS = N_HT // (NW * CHUNK)
Q_CHUNKS = N_Q // (NW * CHUNK)

GROUPS = CHUNK // 16

_LANE = None


def _fix_padding(idx_ref, j, rows_ref):
    zeros16 = jnp.zeros((16,), jnp.float32)

    def group_body(g, carry):
        idx16 = idx_ref[pl.ds(j * CHUNK + g * 16, 16)]
        is_zero = idx16 == 0
        nzero = plsc.all_reduce_population_count(is_zero)

        @pl.when(nzero[0] > 0)
        def _():
            lane = lax.iota(jnp.int32, 16)
            for jj in range(16):
                nj = plsc.all_reduce_population_count(is_zero & (lane == jj))

                @pl.when(nj[0] > 0)
                def _():
                    row = g * 16 + jj
                    for c in range(D // 16):
                        rows_ref[row, pl.ds(c * 16, 16)] = zeros16

        return carry

    lax.fori_loop(0, GROUPS, group_body, 0)


def _body(ent_hbm, rel_hbm, hti_hbm, qri_hbm, qei_hbm,
          out_ht, out_qr, out_qe, idx_v, rows_v, sem_g, sem_w):
    wid = lax.axis_index("s") * NC + lax.axis_index("c")

    for (tab, idx_hbm, out_hbm, nch) in (
        (ent_hbm, hti_hbm, out_ht, HT_CHUNKS),
        (rel_hbm, qri_hbm, out_qr, Q_CHUNKS),
        (ent_hbm, qei_hbm, out_qe, Q_CHUNKS),
    ):
        pw = nch * CHUNK
        pltpu.sync_copy(idx_hbm.at[pl.ds(wid * pw, pw)],
                        idx_v.at[pl.ds(0, pw)])

        def gather(j, b, tab=tab):
            pltpu.async_copy(tab.at[idx_v.at[pl.ds(j * CHUNK, CHUNK)]],
                             rows_v.at[b], sem_g)

        def start_write(j, b, out_hbm=out_hbm, nch=nch):
            base = (wid * nch + j) * CHUNK
            pltpu.async_copy(rows_v.at[b], out_hbm.at[pl.ds(base, CHUNK)],
                             sem_w)

        def wait_write(j, b, out_hbm=out_hbm, nch=nch):
            base = (wid * nch + j) * CHUNK
            pltpu.make_async_copy(rows_v.at[b],
                                  out_hbm.at[pl.ds(base, CHUNK)],
                                  sem_w).wait()

        def wait_gather(j, b, tab=tab):
            pltpu.make_async_copy(tab.at[idx_v.at[pl.ds(j * CHUNK, CHUNK)]],
                                  rows_v.at[b], sem_g).wait()

        for k in range(NBUF - 1):
            gather(k, k)

        def chunk_body(j, carry):
            b = j % NBUF
            wait_gather(j, b)

            @pl.when(j >= 1)
            def _():
                wait_write(j - 1, (j - 1) % NBUF)

            @pl.when(j + NBUF - 1 < nch)
            def _():
                gather(j + NBUF - 1, (j + NBUF - 1) % NBUF)

            _fix_padding(idx_v, j, rows_v.at[b])
            start_write(j, b)
            return carry

        lax.fori_loop(0, nch, chunk_body, 0)
        wait_write(nch - 1, (nch - 1) % NBUF)


@jax.jit
def _run(ht_flat, qrel_flat, qent_flat, ent_embedding, rel_embedding):
    mesh = plsc.VectorSubcoreMesh(core_axis_name="c", subcore_axis_name="s",
                                  num_cores=NC, num_subcores=NS)
    out_type = (
        jax.ShapeDtypeStruct((N_HT, D), jnp.float32),
        jax.ShapeDtypeStruct((N_Q, D), jnp.float32),
        jax.ShapeDtypeStruct((N_Q, D), jnp.float32),
    )
    scratch = [
        pltpu.VMEM((Q_CHUNKS * CHUNK,), jnp.int32),
        pltpu.VMEM((NBUF, CHUNK, D), jnp.float32),
        pltpu.SemaphoreType.DMA,
        pltpu.SemaphoreType.DMA,
    ]
    f = pl.kernel(_body, out_type=out_type, mesh=mesh, scratch_types=scratch,
                  compiler_params=pltpu.CompilerParams(
                      needs_layout_passes=False))
    return f(ent_embedding, rel_embedding, ht_flat, qrel_flat, qent_flat)


def kernel(ht_idx, qual_idx, ent_embedding, rel_embedding):
    ht_flat = ht_idx.astype(jnp.int32).reshape(N_HT)
    qual = qual_idx.astype(jnp.int32)
    qrel_flat = qual[:, :, ::2].reshape(N_Q)
    qent_flat = qual[:, :, 1::2].reshape(N_Q)
    out_ht, out_qr, out_qe = _run(ht_flat, qrel_flat, qent_flat,
                                  ent_embedding, rel_embedding)
    return (out_ht.reshape(B, T, 2, D),
            out_qr.reshape(B, T, QUAL // 2, D),
            out_qe.reshape(B, T, QUAL // 2, D))


# speedup vs baseline: 2.7903x; 1.1145x over previous
import functools

import jax
import jax.numpy as jnp
from jax import lax
from jax.experimental import pallas as pl
from jax.experimental.pallas import tpu as pltpu
from jax.experimental.pallas import tpu_sc as plsc

NUM_ENT = 100000
NUM_REL = 500
D = 128
B = 4096
T = 5
QUAL = 8

NC = 2
NS = 16
NW = NC * NS

CHUNK = 128
NBUF = 4

N_HT = B * T * 2
N_Q = B * T * (QUAL // 2)

HT_CHUNKS = N_HT // (NW * CHUNK)
Q_CHUNKS = N_Q // (NW * CHUNK)
Q_PW = Q_CHUNKS * CHUNK

GROUPS = CHUNK // 16

_LANE = None


def _fix_padding(idx_ref, j, rows_ref):
    zeros16 = jnp.zeros((16,), jnp.float32)

    def group_body(g, carry):
        idx16 = idx_ref[pl.ds(j * CHUNK + g * 16, 16)]
        is_zero = idx16 == 0
        nzero = plsc.all_reduce_population_count(is_zero)

        @pl.when(nzero[0] > 0)
        def _():
            lane = lax.iota(jnp.int32, 16)
            for jj in range(16):
                nj = plsc.all_reduce_population_count(is_zero & (lane == jj))

                @pl.when(nj[0] > 0)
                def _():
                    row = g * 16 + jj
                    for c in range(D // 16):
                        rows_ref[row, pl.ds(c * 16, 16)] = zeros16

        return carry

    lax.fori_loop(0, GROUPS, group_body, 0)


def _deinterleave(qint_v, idx_v, phase, n):
    def dbody(k, carry):
        lane = lax.iota(jnp.int32, 16)
        src = (k * 16 + lane) * 2 + phase
        idx_v[pl.ds(k * 16, 16)] = plsc.load_gather(qint_v, [src])
        return carry

    lax.fori_loop(0, n // 16, dbody, 0)


def _body(ent_hbm, rel_hbm, hti_hbm, q_hbm,
          out_ht, out_qr, out_qe, qint_v, idx_v, rows_v, sem_g, sem_w):
    wid = lax.axis_index("s") * NC + lax.axis_index("c")

    pltpu.sync_copy(q_hbm.at[pl.ds(wid * 2 * Q_PW, 2 * Q_PW)], qint_v)

    for (tab, seg, out_hbm, nch) in (
        (ent_hbm, 0, out_ht, HT_CHUNKS),
        (rel_hbm, 1, out_qr, Q_CHUNKS),
        (ent_hbm, 2, out_qe, Q_CHUNKS),
    ):
        pw = nch * CHUNK
        if seg == 0:
            pltpu.sync_copy(hti_hbm.at[pl.ds(wid * pw, pw)],
                            idx_v.at[pl.ds(0, pw)])
        else:
            _deinterleave(qint_v, idx_v, seg - 1, pw)

        def gather(j, b, tab=tab):
            pltpu.async_copy(tab.at[idx_v.at[pl.ds(j * CHUNK, CHUNK)]],
                             rows_v.at[b], sem_g)

        def start_write(j, b, out_hbm=out_hbm, nch=nch):
            base = (wid * nch + j) * CHUNK
            pltpu.async_copy(rows_v.at[b], out_hbm.at[pl.ds(base, CHUNK)],
                             sem_w)

        def wait_write(j, b, out_hbm=out_hbm, nch=nch):
            base = (wid * nch + j) * CHUNK
            pltpu.make_async_copy(rows_v.at[b],
                                  out_hbm.at[pl.ds(base, CHUNK)],
                                  sem_w).wait()

        def wait_gather(j, b, tab=tab):
            pltpu.make_async_copy(tab.at[idx_v.at[pl.ds(j * CHUNK, CHUNK)]],
                                  rows_v.at[b], sem_g).wait()

        for k in range(NBUF - 1):
            gather(k, k)

        def chunk_body(j, carry):
            b = j % NBUF
            wait_gather(j, b)

            @pl.when(j >= 1)
            def _():
                wait_write(j - 1, (j - 1) % NBUF)

            @pl.when(j + NBUF - 1 < nch)
            def _():
                gather(j + NBUF - 1, (j + NBUF - 1) % NBUF)

            _fix_padding(idx_v, j, rows_v.at[b])
            start_write(j, b)
            return carry

        lax.fori_loop(0, nch, chunk_body, 0)
        wait_write(nch - 1, (nch - 1) % NBUF)


@jax.jit
def _run(ht_flat, q_flat, ent_embedding, rel_embedding):
    mesh = plsc.VectorSubcoreMesh(core_axis_name="c", subcore_axis_name="s",
                                  num_cores=NC, num_subcores=NS)
    out_type = (
        jax.ShapeDtypeStruct((N_HT, D), jnp.float32),
        jax.ShapeDtypeStruct((N_Q, D), jnp.float32),
        jax.ShapeDtypeStruct((N_Q, D), jnp.float32),
    )
    scratch = [
        pltpu.VMEM((2 * Q_PW,), jnp.int32),
        pltpu.VMEM((Q_PW,), jnp.int32),
        pltpu.VMEM((NBUF, CHUNK, D), jnp.float32),
        pltpu.SemaphoreType.DMA,
        pltpu.SemaphoreType.DMA,
    ]
    f = pl.kernel(_body, out_type=out_type, mesh=mesh, scratch_types=scratch,
                  compiler_params=pltpu.CompilerParams(
                      needs_layout_passes=False))
    return f(ent_embedding, rel_embedding, ht_flat, q_flat)


def kernel(ht_idx, qual_idx, ent_embedding, rel_embedding):
    ht_flat = ht_idx.astype(jnp.int32).reshape(N_HT)
    q_flat = qual_idx.astype(jnp.int32).reshape(2 * N_Q)
    out_ht, out_qr, out_qe = _run(ht_flat, q_flat,
                                  ent_embedding, rel_embedding)
    return (out_ht.reshape(B, T, 2, D),
            out_qr.reshape(B, T, QUAL // 2, D),
            out_qe.reshape(B, T, QUAL // 2, D))


# speedup vs baseline: 2.8151x; 1.0089x over previous
import functools

import jax
import jax.numpy as jnp
from jax import lax
from jax.experimental import pallas as pl
from jax.experimental.pallas import tpu as pltpu
from jax.experimental.pallas import tpu_sc as plsc

NUM_ENT = 100000
NUM_REL = 500
D = 128
B = 4096
T = 5
QUAL = 8

NC = 2
NS = 16
NW = NC * NS

CHUNK = 128
NBUF = 4

N_HT = B * T * 2
N_Q = B * T * (QUAL // 2)

HT_CHUNKS = N_HT // (NW * CHUNK)
Q_CHUNKS = N_Q // (NW * CHUNK)
Q_PW = Q_CHUNKS * CHUNK

GROUPS = CHUNK // 16

_LANE = None


def _fix_padding(idx_ref, j, rows_ref):
    zeros16 = jnp.zeros((16,), jnp.float32)

    def group_body(g, carry):
        idx16 = idx_ref[pl.ds(j * CHUNK + g * 16, 16)]
        is_zero = idx16 == 0
        nzero = plsc.all_reduce_population_count(is_zero)

        @pl.when(nzero[0] > 0)
        def _():
            lane = lax.iota(jnp.int32, 16)
            for jj in range(16):
                nj = plsc.all_reduce_population_count(is_zero & (lane == jj))

                @pl.when(nj[0] > 0)
                def _():
                    row = g * 16 + jj
                    for c in range(D // 16):
                        rows_ref[row, pl.ds(c * 16, 16)] = zeros16

        return carry

    lax.fori_loop(0, GROUPS, group_body, 0)


def _deinterleave(qint_v, idx_v, phase, n):
    def dbody(k, carry):
        lane = lax.iota(jnp.int32, 16)
        src = (k * 16 + lane) * 2 + phase
        idx_v[pl.ds(k * 16, 16)] = plsc.load_gather(qint_v, [src])
        return carry

    lax.fori_loop(0, n // 16, dbody, 0)


def _body(ent_hbm, rel_hbm, comb_hbm,
          out_ht, out_qr, out_qe, qint_v, idx_v, rows_v, sem_g, sem_w):
    wid = lax.axis_index("s") * NC + lax.axis_index("c")

    pltpu.sync_copy(comb_hbm.at[pl.ds(N_HT + wid * 2 * Q_PW, 2 * Q_PW)],
                    qint_v)

    for (tab, seg, out_hbm, nch) in (
        (ent_hbm, 0, out_ht, HT_CHUNKS),
        (rel_hbm, 1, out_qr, Q_CHUNKS),
        (ent_hbm, 2, out_qe, Q_CHUNKS),
    ):
        pw = nch * CHUNK
        if seg == 0:
            pltpu.sync_copy(comb_hbm.at[pl.ds(wid * pw, pw)],
                            idx_v.at[pl.ds(0, pw)])
        else:
            _deinterleave(qint_v, idx_v, seg - 1, pw)

        def gather(j, b, tab=tab):
            pltpu.async_copy(tab.at[idx_v.at[pl.ds(j * CHUNK, CHUNK)]],
                             rows_v.at[b], sem_g)

        def start_write(j, b, out_hbm=out_hbm, nch=nch):
            base = (wid * nch + j) * CHUNK
            pltpu.async_copy(rows_v.at[b], out_hbm.at[pl.ds(base, CHUNK)],
                             sem_w)

        def wait_write(j, b, out_hbm=out_hbm, nch=nch):
            base = (wid * nch + j) * CHUNK
            pltpu.make_async_copy(rows_v.at[b],
                                  out_hbm.at[pl.ds(base, CHUNK)],
                                  sem_w).wait()

        def wait_gather(j, b, tab=tab):
            pltpu.make_async_copy(tab.at[idx_v.at[pl.ds(j * CHUNK, CHUNK)]],
                                  rows_v.at[b], sem_g).wait()

        for k in range(NBUF - 1):
            gather(k, k)

        def chunk_body(j, carry):
            b = j % NBUF
            wait_gather(j, b)

            @pl.when(j >= 1)
            def _():
                wait_write(j - 1, (j - 1) % NBUF)

            @pl.when(j + NBUF - 1 < nch)
            def _():
                gather(j + NBUF - 1, (j + NBUF - 1) % NBUF)

            _fix_padding(idx_v, j, rows_v.at[b])
            start_write(j, b)
            return carry

        lax.fori_loop(0, nch, chunk_body, 0)
        wait_write(nch - 1, (nch - 1) % NBUF)


@jax.jit
def _run(comb_idx, ent_embedding, rel_embedding):
    mesh = plsc.VectorSubcoreMesh(core_axis_name="c", subcore_axis_name="s",
                                  num_cores=NC, num_subcores=NS)
    out_type = (
        jax.ShapeDtypeStruct((N_HT, D), jnp.float32),
        jax.ShapeDtypeStruct((N_Q, D), jnp.float32),
        jax.ShapeDtypeStruct((N_Q, D), jnp.float32),
    )
    scratch = [
        pltpu.VMEM((2 * Q_PW,), jnp.int32),
        pltpu.VMEM((Q_PW,), jnp.int32),
        pltpu.VMEM((NBUF, CHUNK, D), jnp.float32),
        pltpu.SemaphoreType.DMA,
        pltpu.SemaphoreType.DMA,
    ]
    f = pl.kernel(_body, out_type=out_type, mesh=mesh, scratch_types=scratch,
                  compiler_params=pltpu.CompilerParams(
                      needs_layout_passes=False))
    return f(ent_embedding, rel_embedding, comb_idx)


def kernel(ht_idx, qual_idx, ent_embedding, rel_embedding):
    comb_idx = jnp.concatenate(
        [ht_idx.astype(jnp.int32).reshape(N_HT),
         qual_idx.astype(jnp.int32).reshape(2 * N_Q)])
    out_ht, out_qr, out_qe = _run(comb_idx, ent_embedding, rel_embedding)
    return (out_ht.reshape(B, T, 2, D),
            out_qr.reshape(B, T, QUAL // 2, D),
            out_qe.reshape(B, T, QUAL // 2, D))
